# Initial kernel scaffold; baseline (speedup 1.0000x reference)
import jax
import jax.numpy as jnp
from jax.experimental import pallas as pl


def kernel(x, edge_index, gamma, beta):
    raise NotImplementedError("write your pallas kernel here")


# speedup vs baseline: 2.8742x; 2.8742x over previous
import functools

import jax
import jax.numpy as jnp
from jax import lax
from jax.experimental import pallas as pl
from jax.experimental.pallas import tpu as pltpu
from jax.experimental.pallas import tpu_sc as plsc

N = 10000
D = 128
E = 320000
NUM_LAYERS = 2
BN_EPS = 1e-5

NC = 2
NS = 16
NW = NC * NS
LANES = 16
G = D // LANES

ROWS_W = 320
NPAD = NW * ROWS_W
HALF = ROWS_W // 2
TRASH = NPAD
ACC_ROWS = 10496
ZROWS = ACC_ROWS // NS

CH = 128
KCH = 80
EP = NW * KCH * CH

_mesh = plsc.VectorSubcoreMesh(
    core_axis_name="c", subcore_axis_name="s", num_cores=NC, num_subcores=NS
)


def _wid():
    return lax.axis_index("s") * NC + lax.axis_index("c")


@functools.partial(
    pl.kernel,
    out_type=jax.ShapeDtypeStruct((NC, NPAD, D), jnp.float32),
    mesh=_mesh,
    scratch_types=[
        pltpu.VMEM_SHARED((ACC_ROWS, D), jnp.float32),
        pltpu.VMEM((KCH, CH), jnp.int32),
        pltpu.VMEM((KCH, CH), jnp.int32),
        pltpu.VMEM((CH, D), jnp.float32),
        pltpu.SemaphoreType.DMA,
    ],
)
def _scatter(h_hbm, src_hbm, dst_hbm, out_hbm, acc_sh, src_v, dst_v, rows_v, sem):
    c = lax.axis_index("c")
    s = lax.axis_index("s")
    w = _wid()

    zero = jnp.zeros((LANES,), jnp.float32)

    @pl.loop(0, CH)
    def _(i):
        for j in range(G):
            rows_v[i, pl.ds(j * LANES, LANES)] = zero

    zbase = s * ZROWS
    nfull = ZROWS // CH
    for k in range(nfull):
        pltpu.sync_copy(rows_v, acc_sh.at[pl.ds(zbase + k * CH, CH)])
    rem = ZROWS - nfull * CH
    if rem:
        pltpu.sync_copy(
            rows_v.at[pl.ds(0, rem)],
            acc_sh.at[pl.ds(zbase + nfull * CH, rem)],
        )
    plsc.subcore_barrier()

    pltpu.sync_copy(src_hbm.at[pl.ds(w * KCH, KCH)], src_v)
    pltpu.sync_copy(dst_hbm.at[pl.ds(w * KCH, KCH)], dst_v)

    @pl.loop(0, KCH)
    def _(g):
        pltpu.async_copy(h_hbm.at[src_v.at[g]], rows_v, sem).wait()
        pltpu.sync_copy(rows_v, acc_sh.at[dst_v.at[g]], add=True)

    plsc.subcore_barrier()
    rows_out = NPAD // NS
    pltpu.sync_copy(
        acc_sh.at[pl.ds(s * rows_out, rows_out)],
        out_hbm.at[c, pl.ds(s * rows_out, rows_out)],
    )


@functools.partial(
    pl.kernel,
    out_type=(
        jax.ShapeDtypeStruct((NPAD, D), jnp.float32),
        jax.ShapeDtypeStruct((NW, 2 * G, LANES), jnp.float32),
    ),
    mesh=_mesh,
    scratch_types=[
        pltpu.VMEM((HALF, D), jnp.float32),
        pltpu.VMEM((HALF, D), jnp.float32),
        pltpu.VMEM((HALF, D), jnp.float32),
        pltpu.VMEM((2 * G, LANES), jnp.float32),
    ],
)
def _combine(h_hbm, p_hbm, v_hbm, stats_hbm, hbuf, p0buf, p1buf, stats_v):
    w = _wid()
    zero = jnp.zeros((LANES,), jnp.float32)
    for j in range(2 * G):
        stats_v[j, pl.ds(0, LANES)] = zero

    for half in range(2):
        r0 = w * ROWS_W + half * HALF
        pltpu.sync_copy(h_hbm.at[pl.ds(r0, HALF)], hbuf)
        pltpu.sync_copy(p_hbm.at[0, pl.ds(r0, HALF)], p0buf)
        pltpu.sync_copy(p_hbm.at[1, pl.ds(r0, HALF)], p1buf)

        @pl.loop(0, HALF)
        def _(r):
            m = jnp.where(r0 + r < N, 1.0, 0.0).astype(jnp.float32)
            for j in range(G):
                sl = pl.ds(j * LANES, LANES)
                val = hbuf[r, sl] + p0buf[r, sl] + p1buf[r, sl]
                hbuf[r, sl] = val
                vm = val * m
                plsc.addupdate(stats_v.at[j], vm)
                plsc.addupdate(stats_v.at[G + j], vm * val)

        pltpu.sync_copy(hbuf, v_hbm.at[pl.ds(r0, HALF)])

    pltpu.sync_copy(stats_v, stats_hbm.at[w])


@functools.partial(
    pl.kernel,
    out_type=jax.ShapeDtypeStruct((NPAD, D), jnp.float32),
    mesh=_mesh,
    scratch_types=[
        pltpu.VMEM((NW, 2 * G, LANES), jnp.float32),
        pltpu.VMEM((D,), jnp.float32),
        pltpu.VMEM((D,), jnp.float32),
        pltpu.VMEM((2 * G, LANES), jnp.float32),
        pltpu.VMEM((HALF, D), jnp.float32),
    ],
)
def _normalize(v_hbm, stats_hbm, g_hbm, b_hbm, out_hbm, sbuf, gbuf, bbuf, ab, vbuf):
    w = _wid()
    pltpu.sync_copy(stats_hbm, sbuf)
    pltpu.sync_copy(g_hbm, gbuf)
    pltpu.sync_copy(b_hbm, bbuf)

    inv_n = jnp.float32(1.0 / N)
    for j in range(G):
        ssum = jnp.zeros((LANES,), jnp.float32)
        ssq = jnp.zeros((LANES,), jnp.float32)
        (ssum, ssq) = pl.loop(0, NW, init_carry=(ssum, ssq))(
            lambda w2, carry, _j=j: (carry[0] + sbuf[w2, _j], carry[1] + sbuf[w2, G + _j])
        )
        mean = ssum * inv_n
        var = ssq * inv_n - mean * mean
        z = var + jnp.float32(BN_EPS)
        y = (z + jnp.float32(1.0)) * jnp.float32(0.5)
        for _ in range(40):
            y = (y + z / y) * jnp.float32(0.5)
        sl = pl.ds(j * LANES, LANES)
        a = gbuf[sl] / y
        b = bbuf[sl] - mean * a
        ab[j, pl.ds(0, LANES)] = a
        ab[G + j, pl.ds(0, LANES)] = b

    for half in range(2):
        r0 = w * ROWS_W + half * HALF
        pltpu.sync_copy(v_hbm.at[pl.ds(r0, HALF)], vbuf)

        @pl.loop(0, HALF)
        def _(r):
            for j in range(G):
                sl = pl.ds(j * LANES, LANES)
                a = ab[j, pl.ds(0, LANES)]
                b = ab[G + j, pl.ds(0, LANES)]
                vbuf[r, sl] = vbuf[r, sl] * a + b

        pltpu.sync_copy(vbuf, out_hbm.at[pl.ds(r0, HALF)])


def kernel(x, edge_index, gamma, beta):
    src = edge_index[0]
    dst = edge_index[1]
    pad_e = EP - E
    src_p = jnp.concatenate([src, jnp.zeros((pad_e,), jnp.int32)]).reshape(
        NW * KCH, CH
    )
    dst_p = jnp.concatenate([dst, jnp.full((pad_e,), TRASH, jnp.int32)]).reshape(
        NW * KCH, CH
    )
    h = jnp.concatenate([x, jnp.zeros((NPAD - N, D), jnp.float32)], axis=0)
    for l in range(NUM_LAYERS):
        partials = _scatter(h, src_p, dst_p)
        v, stats = _combine(h, partials)
        h = _normalize(v, stats, gamma[l], beta[l])
    return h[:N]


# speedup vs baseline: 2.9941x; 1.0417x over previous
import functools

import jax
import jax.numpy as jnp
from jax import lax
from jax.experimental import pallas as pl
from jax.experimental.pallas import tpu as pltpu
from jax.experimental.pallas import tpu_sc as plsc

N = 10000
D = 128
E = 320000
NUM_LAYERS = 2
BN_EPS = 1e-5

NC = 2
NS = 16
NW = NC * NS
LANES = 16
G = D // LANES

ROWS_W = 320
NPAD = NW * ROWS_W
HALF = ROWS_W // 2
TRASH = NPAD
ACC_ROWS = 10368
ZROWS = ACC_ROWS // NS

CH = 64
KCH = 160
EP = NW * KCH * CH

_mesh = plsc.VectorSubcoreMesh(
    core_axis_name="c", subcore_axis_name="s", num_cores=NC, num_subcores=NS
)


def _wid():
    return lax.axis_index("s") * NC + lax.axis_index("c")


@functools.partial(
    pl.kernel,
    out_type=jax.ShapeDtypeStruct((NC, NPAD, D), jnp.float32),
    mesh=_mesh,
    scratch_types=[
        pltpu.VMEM_SHARED((ACC_ROWS, D), jnp.float32),
        pltpu.VMEM((KCH // 2, CH), jnp.int32),
        pltpu.VMEM((KCH // 2, CH), jnp.int32),
        pltpu.VMEM((CH, D), jnp.float32),
        pltpu.VMEM((CH, D), jnp.float32),
        pltpu.SemaphoreType.DMA,
        pltpu.SemaphoreType.DMA,
    ],
)
def _scatter(
    h_hbm, src_hbm, dst_hbm, out_hbm, acc_sh, src_v, dst_v, rows_v, rows_v1, sem, sem1
):
    c = lax.axis_index("c")
    s = lax.axis_index("s")
    w = _wid()
    kch2 = KCH // 2

    zero = jnp.zeros((LANES,), jnp.float32)

    @pl.loop(0, CH)
    def _(i):
        for j in range(G):
            rows_v[i, pl.ds(j * LANES, LANES)] = zero

    zbase = s * (NPAD // NS)
    for k in range(NPAD // NS // CH):
        pltpu.sync_copy(rows_v, acc_sh.at[pl.ds(zbase + k * CH, CH)])
    plsc.subcore_barrier()

    def _wait(buf, sm):
        pltpu.make_async_copy(h_hbm.at[src_v.at[0]], buf, sm).wait()

    for phase in range(2):
        base = w * KCH + phase * kch2
        pltpu.sync_copy(src_hbm.at[pl.ds(base, kch2)], src_v)
        pltpu.sync_copy(dst_hbm.at[pl.ds(base, kch2)], dst_v)
        pltpu.async_copy(h_hbm.at[src_v.at[0]], rows_v, sem)

        @pl.loop(0, kch2 // 2)
        def _(p):
            g = p * 2
            pltpu.async_copy(h_hbm.at[src_v.at[g + 1]], rows_v1, sem1)
            _wait(rows_v, sem)
            pltpu.sync_copy(rows_v, acc_sh.at[dst_v.at[g]], add=True)
            gn = jnp.minimum(g + 2, kch2 - 1)
            pltpu.async_copy(h_hbm.at[src_v.at[gn]], rows_v, sem)
            _wait(rows_v1, sem1)
            pltpu.sync_copy(rows_v1, acc_sh.at[dst_v.at[g + 1]], add=True)

        _wait(rows_v, sem)

    plsc.subcore_barrier()
    rows_out = NPAD // NS
    pltpu.sync_copy(
        acc_sh.at[pl.ds(s * rows_out, rows_out)],
        out_hbm.at[c, pl.ds(s * rows_out, rows_out)],
    )


@functools.partial(
    pl.kernel,
    out_type=(
        jax.ShapeDtypeStruct((NPAD, D), jnp.float32),
        jax.ShapeDtypeStruct((NW, 2 * G, LANES), jnp.float32),
    ),
    mesh=_mesh,
    scratch_types=[
        pltpu.VMEM((HALF, D), jnp.float32),
        pltpu.VMEM((HALF, D), jnp.float32),
        pltpu.VMEM((HALF, D), jnp.float32),
        pltpu.VMEM((2 * G, LANES), jnp.float32),
    ],
)
def _combine(h_hbm, p_hbm, v_hbm, stats_hbm, hbuf, p0buf, p1buf, stats_v):
    w = _wid()
    zero = jnp.zeros((LANES,), jnp.float32)
    for j in range(2 * G):
        stats_v[j, pl.ds(0, LANES)] = zero

    for half in range(2):
        r0 = w * ROWS_W + half * HALF
        pltpu.sync_copy(h_hbm.at[pl.ds(r0, HALF)], hbuf)
        pltpu.sync_copy(p_hbm.at[0, pl.ds(r0, HALF)], p0buf)
        pltpu.sync_copy(p_hbm.at[1, pl.ds(r0, HALF)], p1buf)

        @pl.loop(0, HALF)
        def _(r):
            m = jnp.where(r0 + r < N, 1.0, 0.0).astype(jnp.float32)
            for j in range(G):
                sl = pl.ds(j * LANES, LANES)
                val = hbuf[r, sl] + p0buf[r, sl] + p1buf[r, sl]
                hbuf[r, sl] = val
                vm = val * m
                plsc.addupdate(stats_v.at[j], vm)
                plsc.addupdate(stats_v.at[G + j], vm * val)

        pltpu.sync_copy(hbuf, v_hbm.at[pl.ds(r0, HALF)])

    pltpu.sync_copy(stats_v, stats_hbm.at[w])


@functools.partial(
    pl.kernel,
    out_type=jax.ShapeDtypeStruct((NPAD, D), jnp.float32),
    mesh=_mesh,
    scratch_types=[
        pltpu.VMEM((NW, 2 * G, LANES), jnp.float32),
        pltpu.VMEM((D,), jnp.float32),
        pltpu.VMEM((D,), jnp.float32),
        pltpu.VMEM((2 * G, LANES), jnp.float32),
        pltpu.VMEM((HALF, D), jnp.float32),
    ],
)
def _normalize(v_hbm, stats_hbm, g_hbm, b_hbm, out_hbm, sbuf, gbuf, bbuf, ab, vbuf):
    w = _wid()
    pltpu.sync_copy(stats_hbm, sbuf)
    pltpu.sync_copy(g_hbm, gbuf)
    pltpu.sync_copy(b_hbm, bbuf)

    inv_n = jnp.float32(1.0 / N)
    for j in range(G):
        ssum = jnp.zeros((LANES,), jnp.float32)
        ssq = jnp.zeros((LANES,), jnp.float32)
        (ssum, ssq) = pl.loop(0, NW, init_carry=(ssum, ssq))(
            lambda w2, carry, _j=j: (carry[0] + sbuf[w2, _j], carry[1] + sbuf[w2, G + _j])
        )
        mean = ssum * inv_n
        var = ssq * inv_n - mean * mean
        z = var + jnp.float32(BN_EPS)
        y = (z + jnp.float32(1.0)) * jnp.float32(0.5)
        for _ in range(40):
            y = (y + z / y) * jnp.float32(0.5)
        sl = pl.ds(j * LANES, LANES)
        a = gbuf[sl] / y
        b = bbuf[sl] - mean * a
        ab[j, pl.ds(0, LANES)] = a
        ab[G + j, pl.ds(0, LANES)] = b

    for half in range(2):
        r0 = w * ROWS_W + half * HALF
        pltpu.sync_copy(v_hbm.at[pl.ds(r0, HALF)], vbuf)

        @pl.loop(0, HALF)
        def _(r):
            for j in range(G):
                sl = pl.ds(j * LANES, LANES)
                a = ab[j, pl.ds(0, LANES)]
                b = ab[G + j, pl.ds(0, LANES)]
                vbuf[r, sl] = vbuf[r, sl] * a + b

        pltpu.sync_copy(vbuf, out_hbm.at[pl.ds(r0, HALF)])


def kernel(x, edge_index, gamma, beta):
    src = edge_index[0]
    dst = edge_index[1]
    pad_e = EP - E
    src_p = jnp.concatenate([src, jnp.zeros((pad_e,), jnp.int32)]).reshape(
        NW * KCH, CH
    )
    dst_p = jnp.concatenate([dst, jnp.full((pad_e,), TRASH, jnp.int32)]).reshape(
        NW * KCH, CH
    )
    h = jnp.concatenate([x, jnp.zeros((NPAD - N, D), jnp.float32)], axis=0)
    for l in range(NUM_LAYERS):
        partials = _scatter(h, src_p, dst_p)
        v, stats = _combine(h, partials)
        h = _normalize(v, stats, gamma[l], beta[l])
    return h[:N]
